# Initial kernel scaffold; baseline (speedup 1.0000x reference)
import jax
import jax.numpy as jnp
from jax.experimental import pallas as pl


def kernel(x, gamma, beta):
    raise NotImplementedError("write your pallas kernel here")


# speedup vs baseline: 16.5777x; 16.5777x over previous
import math

import jax
import jax.numpy as jnp
from jax.experimental import pallas as pl
from jax.experimental.pallas import tpu as pltpu

_EPS = 1e-5


def _sln_body(x_ref, gamma_ref, beta_ref, o_ref):
    z = x_ref[...]
    ts, lanes = z.shape
    dd = lanes // 2
    lane = jax.lax.broadcasted_iota(jnp.int32, (1, lanes), 1)
    m = (lane % 2 == 0).astype(z.dtype)
    w = 1.0 - m

    zr = pltpu.roll(z, lanes - 1, axis=1)
    zm = z * m
    zsq = z * z

    s_r = jnp.sum(zm, axis=1, keepdims=True)
    s_all = jnp.sum(z, axis=1, keepdims=True)
    s_i = s_all - s_r
    s_rr = jnp.sum(zsq * m, axis=1, keepdims=True)
    s_sq = jnp.sum(zsq, axis=1, keepdims=True)
    s_ii = s_sq - s_rr
    s_ri = jnp.sum(zm * zr, axis=1, keepdims=True)

    d_f = jnp.float32(dd)
    inv_dm1 = 1.0 / (d_f - 1.0)
    mr = s_r / d_f
    mi = s_i / d_f
    a = (s_rr - d_f * mr * mr) * inv_dm1 + _EPS
    c = (s_ii - d_f * mi * mi) * inv_dm1 + _EPS
    b = (s_ri - d_f * mr * mi) * inv_dm1

    t = a + c
    s = jnp.sqrt(a * c - b * b)
    denom_inv = jax.lax.rsqrt(t + 2.0 * s) / s
    w00 = (c + s) * denom_inv
    w11 = (a + s) * denom_inv
    w01 = -b * denom_inv

    g0 = gamma_ref[:, 0:1]
    g1 = gamma_ref[:, 1:2]
    g2 = gamma_ref[:, 2:3]
    m00 = g0 * w00 + g1 * w01
    m01 = g0 * w01 + g1 * w11
    m10 = g1 * w00 + g2 * w01
    m11 = g1 * w01 + g2 * w11

    off_e = -(m00 * mr + m01 * mi)
    off_o = -(m10 * mr + m11 * mi)

    zl = pltpu.roll(z, 1, axis=1)
    y = ((m * m00 + w * m11) * z
         + (m * m01) * zr
         + (w * m10) * zl
         + beta_ref[...]
         + m * off_e + w * off_o)
    o_ref[...] = y


def kernel(x, gamma, beta):
    bb, ss, dd, _ = x.shape
    bs = bb * ss
    lanes = 2 * dd
    x2 = x.reshape(bs, lanes)
    ts = math.gcd(256, math.gcd(bs, dd))
    gperiod = dd // ts

    out = pl.pallas_call(
        _sln_body,
        grid=(bs // ts,),
        in_specs=[
            pl.BlockSpec((ts, lanes), lambda j: (j, 0)),
            pl.BlockSpec((ts, 3), lambda j: (j % gperiod, 0)),
            pl.BlockSpec((1, lanes), lambda j: (0, 0)),
        ],
        out_specs=pl.BlockSpec((ts, lanes), lambda j: (j, 0)),
        out_shape=jax.ShapeDtypeStruct((bs, lanes), x.dtype),
        compiler_params=pltpu.CompilerParams(
            dimension_semantics=("parallel",),
        ),
    )(x2, gamma, beta.reshape(1, lanes))
    return out.reshape(bb, ss, dd, 2)


# speedup vs baseline: 17.7947x; 1.0734x over previous
import math

import jax
import jax.numpy as jnp
from jax.experimental import pallas as pl
from jax.experimental.pallas import tpu as pltpu

_EPS = 1e-5


def _sln_body(x_ref, gamma_ref, beta_ref, o_ref):
    z = x_ref[...]
    ts, lanes = z.shape
    dd = lanes // 2

    lane = jax.lax.broadcasted_iota(jnp.int32, (1, lanes), 1)
    meven = (lane % 2) == 0

    zr = pltpu.roll(z, lanes - 1, axis=1)
    zl = pltpu.roll(z, 1, axis=1)
    zx = jnp.where(meven, zr, zl)

    zsq = z * z
    zc = z * zx

    def _acc128(arr):
        parts = [arr[:, k * 128:(k + 1) * 128] for k in range(lanes // 128)]
        while len(parts) > 1:
            parts = [parts[i] + parts[i + 1] for i in range(0, len(parts), 2)]
        return parts[0]

    s_a = _acc128(z)
    q_a = _acc128(zsq)
    r_a = _acc128(zc)

    lane128 = jax.lax.broadcasted_iota(jnp.int32, (1, 128), 1)
    m128 = ((lane128 % 2) == 0).astype(z.dtype)
    s_r = jnp.sum(s_a * m128, axis=1, keepdims=True)
    s_i = jnp.sum(s_a, axis=1, keepdims=True) - s_r
    s_rr = jnp.sum(q_a * m128, axis=1, keepdims=True)
    s_ii = jnp.sum(q_a, axis=1, keepdims=True) - s_rr
    s_ri = 0.5 * jnp.sum(r_a, axis=1, keepdims=True)

    d_f = jnp.float32(dd)
    inv_dm1 = 1.0 / (d_f - 1.0)
    mr = s_r / d_f
    mi = s_i / d_f
    a = (s_rr - d_f * mr * mr) * inv_dm1 + _EPS
    c = (s_ii - d_f * mi * mi) * inv_dm1 + _EPS
    b = (s_ri - d_f * mr * mi) * inv_dm1

    t = a + c
    s = jnp.sqrt(a * c - b * b)
    denom_inv = jax.lax.rsqrt(t + 2.0 * s) / s
    w00 = (c + s) * denom_inv
    w11 = (a + s) * denom_inv
    w01 = -b * denom_inv

    g0 = gamma_ref[:, 0:1]
    g1 = gamma_ref[:, 1:2]
    g2 = gamma_ref[:, 2:3]
    m00 = g0 * w00 + g1 * w01
    m01 = g0 * w01 + g1 * w11
    m10 = g1 * w00 + g2 * w01
    m11 = g1 * w01 + g2 * w11

    off_e = -(m00 * mr + m01 * mi)
    off_o = -(m10 * mr + m11 * mi)

    p = jnp.where(meven, m00, m11)
    q = jnp.where(meven, m01, m10)
    o = jnp.where(meven, off_e, off_o)
    o_ref[...] = p * z + q * zx + (o + beta_ref[...])


def kernel(x, gamma, beta):
    bb, ss, dd, _ = x.shape
    bs = bb * ss
    lanes = 2 * dd
    x2 = x.reshape(bs, lanes)
    ts = math.gcd(256, math.gcd(bs, dd))
    gperiod = dd // ts

    out = pl.pallas_call(
        _sln_body,
        grid=(bs // ts,),
        in_specs=[
            pl.BlockSpec((ts, lanes), lambda j: (j, 0)),
            pl.BlockSpec((ts, 3), lambda j: (j % gperiod, 0)),
            pl.BlockSpec((1, lanes), lambda j: (0, 0)),
        ],
        out_specs=pl.BlockSpec((ts, lanes), lambda j: (j, 0)),
        out_shape=jax.ShapeDtypeStruct((bs, lanes), x.dtype),
        compiler_params=pltpu.CompilerParams(
            dimension_semantics=("parallel",),
        ),
    )(x2, gamma, beta.reshape(1, lanes))
    return out.reshape(bb, ss, dd, 2)


# speedup vs baseline: 57.1860x; 3.2136x over previous
import math

import jax
import jax.numpy as jnp
from jax.experimental import pallas as pl
from jax.experimental.pallas import tpu as pltpu

_EPS = 1e-5


def _sln_body(x_ref, gamma_ref, beta_ref, o_ref):
    z = x_ref[...]
    ts, rows, _ = z.shape
    dd = rows * 64

    row = jax.lax.broadcasted_iota(jnp.int32, (1, rows, 1), 1)
    meven = (row % 2) == 0

    zdn = pltpu.roll(z, rows - 1, axis=1)
    zup = pltpu.roll(z, 1, axis=1)
    zx = jnp.where(meven, zdn, zup)

    def _acc8(f):
        parts = [f(k) for k in range(rows // 8)]
        while len(parts) > 1:
            parts = [parts[i] + parts[i + 1] for i in range(0, len(parts), 2)]
        return parts[0]

    def _sl(k):
        return z[:, 8 * k:8 * (k + 1), :]

    def _slx(k):
        return zx[:, 8 * k:8 * (k + 1), :]

    s_a = _acc8(_sl)
    q_a = _acc8(lambda k: _sl(k) * _sl(k))
    r_a = _acc8(lambda k: _sl(k) * _slx(k))

    row8 = jax.lax.broadcasted_iota(jnp.int32, (1, 8, 1), 1)
    m8 = ((row8 % 2) == 0).astype(z.dtype)
    s_r = jnp.sum(s_a * m8, axis=(1, 2), keepdims=True)[:, :, 0]
    s_i = jnp.sum(s_a, axis=(1, 2), keepdims=True)[:, :, 0] - s_r
    s_rr = jnp.sum(q_a * m8, axis=(1, 2), keepdims=True)[:, :, 0]
    s_ii = jnp.sum(q_a, axis=(1, 2), keepdims=True)[:, :, 0] - s_rr
    s_ri = 0.5 * jnp.sum(r_a, axis=(1, 2), keepdims=True)[:, :, 0]

    d_f = jnp.float32(dd)
    inv_dm1 = 1.0 / (d_f - 1.0)
    mr = s_r / d_f
    mi = s_i / d_f
    a = (s_rr - d_f * mr * mr) * inv_dm1 + _EPS
    c = (s_ii - d_f * mi * mi) * inv_dm1 + _EPS
    b = (s_ri - d_f * mr * mi) * inv_dm1

    t = a + c
    s = jnp.sqrt(a * c - b * b)
    denom_inv = jax.lax.rsqrt(t + 2.0 * s) / s
    w00 = (c + s) * denom_inv
    w11 = (a + s) * denom_inv
    w01 = -b * denom_inv

    g0 = gamma_ref[:, 0:1]
    g1 = gamma_ref[:, 1:2]
    g2 = gamma_ref[:, 2:3]
    m00 = (g0 * w00 + g1 * w01)[:, :, None]
    m01 = (g0 * w01 + g1 * w11)[:, :, None]
    m10 = (g1 * w00 + g2 * w01)[:, :, None]
    m11 = (g1 * w01 + g2 * w11)[:, :, None]

    off_e = -(m00 * mr[:, :, None] + m01 * mi[:, :, None])
    off_o = -(m10 * mr[:, :, None] + m11 * mi[:, :, None])

    p = jnp.where(meven, m00, m11)
    q = jnp.where(meven, m01, m10)
    o = jnp.where(meven, off_e, off_o)
    o_ref[...] = p * z + q * zx + (o + beta_ref[...])


def kernel(x, gamma, beta):
    bb, ss, dd, _ = x.shape
    bs = bb * ss
    chunks = dd // 128
    rows = 2 * chunks
    xv = (x.reshape(bs, chunks, 128, 2)
           .transpose(0, 1, 3, 2)
           .reshape(bs, rows, 128))
    betav = (beta.reshape(chunks, 128, 2)
                 .transpose(0, 2, 1)
                 .reshape(1, rows, 128))

    ts = math.gcd(256, math.gcd(bs, dd))
    gperiod = dd // ts

    out = pl.pallas_call(
        _sln_body,
        grid=(bs // ts,),
        in_specs=[
            pl.BlockSpec((ts, rows, 128), lambda j: (j, 0, 0)),
            pl.BlockSpec((ts, 3), lambda j: (j % gperiod, 0)),
            pl.BlockSpec((1, rows, 128), lambda j: (0, 0, 0)),
        ],
        out_specs=pl.BlockSpec((ts, rows, 128), lambda j: (j, 0, 0)),
        out_shape=jax.ShapeDtypeStruct((bs, rows, 128), x.dtype),
        compiler_params=pltpu.CompilerParams(
            dimension_semantics=("parallel",),
        ),
    )(xv, gamma, betav)
    return (out.reshape(bs, chunks, 2, 128)
               .transpose(0, 1, 3, 2)
               .reshape(bb, ss, dd, 2))


# speedup vs baseline: 77.2106x; 1.3502x over previous
import math

import jax
import jax.numpy as jnp
from jax.experimental import pallas as pl
from jax.experimental.pallas import tpu as pltpu

_EPS = 1e-5


def _sln_body(x_ref, gamma_ref, beta_ref, o_ref):
    z = x_ref[...]
    ts, rows, _ = z.shape
    dd = rows * 64

    row8 = jax.lax.broadcasted_iota(jnp.int32, (1, 8, 1), 1)
    meven8 = (row8 % 2) == 0

    def _sl(k):
        return z[:, 8 * k:8 * (k + 1), :]

    def _slx(k):
        zk = _sl(k)
        return jnp.where(meven8,
                         pltpu.roll(zk, 7, axis=1),
                         pltpu.roll(zk, 1, axis=1))

    def _acc8(f):
        parts = [f(k) for k in range(rows // 8)]
        while len(parts) > 1:
            parts = [parts[i] + parts[i + 1] for i in range(0, len(parts), 2)]
        return parts[0]

    s_a = _acc8(_sl)
    q_a = _acc8(lambda k: _sl(k) * _sl(k))
    r_a = _acc8(lambda k: _sl(k) * _slx(k))

    m8 = meven8.astype(z.dtype)
    s_r = jnp.sum(s_a * m8, axis=(1, 2), keepdims=True)[:, :, 0]
    s_i = jnp.sum(s_a, axis=(1, 2), keepdims=True)[:, :, 0] - s_r
    s_rr = jnp.sum(q_a * m8, axis=(1, 2), keepdims=True)[:, :, 0]
    s_ii = jnp.sum(q_a, axis=(1, 2), keepdims=True)[:, :, 0] - s_rr
    s_ri = 0.5 * jnp.sum(r_a, axis=(1, 2), keepdims=True)[:, :, 0]

    d_f = jnp.float32(dd)
    inv_dm1 = 1.0 / (d_f - 1.0)
    mr = s_r / d_f
    mi = s_i / d_f
    a = (s_rr - d_f * mr * mr) * inv_dm1 + _EPS
    c = (s_ii - d_f * mi * mi) * inv_dm1 + _EPS
    b = (s_ri - d_f * mr * mi) * inv_dm1

    t = a + c
    s = jnp.sqrt(a * c - b * b)
    denom_inv = jax.lax.rsqrt(t + 2.0 * s) / s
    w00 = (c + s) * denom_inv
    w11 = (a + s) * denom_inv
    w01 = -b * denom_inv

    g0 = gamma_ref[:, 0:1]
    g1 = gamma_ref[:, 1:2]
    g2 = gamma_ref[:, 2:3]
    m00 = (g0 * w00 + g1 * w01)[:, :, None]
    m01 = (g0 * w01 + g1 * w11)[:, :, None]
    m10 = (g1 * w00 + g2 * w01)[:, :, None]
    m11 = (g1 * w01 + g2 * w11)[:, :, None]

    off_e = -(m00 * mr[:, :, None] + m01 * mi[:, :, None])
    off_o = -(m10 * mr[:, :, None] + m11 * mi[:, :, None])

    p = jnp.where(meven8, m00, m11)
    q = jnp.where(meven8, m01, m10)
    o = jnp.where(meven8, off_e, off_o)
    for k in range(rows // 8):
        bk = beta_ref[:, 8 * k:8 * (k + 1), :]
        o_ref[:, 8 * k:8 * (k + 1), :] = p * _sl(k) + q * _slx(k) + (o + bk)


def kernel(x, gamma, beta):
    bb, ss, dd, _ = x.shape
    bs = bb * ss
    chunks = dd // 128
    rows = 2 * chunks
    xv = (x.reshape(bs, chunks, 128, 2)
           .transpose(0, 1, 3, 2)
           .reshape(bs, rows, 128))
    betav = (beta.reshape(chunks, 128, 2)
                 .transpose(0, 2, 1)
                 .reshape(1, rows, 128))

    ts = math.gcd(256, math.gcd(bs, dd))
    gperiod = dd // ts

    out = pl.pallas_call(
        _sln_body,
        grid=(bs // ts,),
        in_specs=[
            pl.BlockSpec((ts, rows, 128), lambda j: (j, 0, 0)),
            pl.BlockSpec((ts, 3), lambda j: (j % gperiod, 0)),
            pl.BlockSpec((1, rows, 128), lambda j: (0, 0, 0)),
        ],
        out_specs=pl.BlockSpec((ts, rows, 128), lambda j: (j, 0, 0)),
        out_shape=jax.ShapeDtypeStruct((bs, rows, 128), x.dtype),
        compiler_params=pltpu.CompilerParams(
            dimension_semantics=("parallel",),
        ),
    )(xv, gamma, betav)
    return (out.reshape(bs, chunks, 2, 128)
               .transpose(0, 1, 3, 2)
               .reshape(bb, ss, dd, 2))


# speedup vs baseline: 77.2205x; 1.0001x over previous
import math

import jax
import jax.numpy as jnp
from jax.experimental import pallas as pl
from jax.experimental.pallas import tpu as pltpu

_EPS = 1e-5


def _sln_body(x_ref, gamma_ref, beta_ref, o_ref):
    z = x_ref[...]
    ts, rows, _ = z.shape
    dd = rows * 64

    row8 = jax.lax.broadcasted_iota(jnp.int32, (1, 8, 1), 1)
    meven8 = (row8 % 2) == 0

    def _sl(k):
        return z[:, 8 * k:8 * (k + 1), :]

    def _swap(zk):
        return jnp.where(meven8,
                         pltpu.roll(zk, 7, axis=1),
                         pltpu.roll(zk, 1, axis=1))

    s_parts, q_parts, r_parts = [], [], []
    for k in range(rows // 8):
        zk = _sl(k)
        zxk = _swap(zk)
        s_parts.append(zk)
        q_parts.append(zk * zk)
        r_parts.append(zk * zxk)

    def _tree(parts):
        while len(parts) > 1:
            parts = [parts[i] + parts[i + 1] for i in range(0, len(parts), 2)]
        return parts[0]

    s_a = _tree(s_parts)
    q_a = _tree(q_parts)
    r_a = _tree(r_parts)

    m8 = meven8.astype(z.dtype)
    s_r = jnp.sum(s_a * m8, axis=(1, 2), keepdims=True)[:, :, 0]
    s_i = jnp.sum(s_a, axis=(1, 2), keepdims=True)[:, :, 0] - s_r
    s_rr = jnp.sum(q_a * m8, axis=(1, 2), keepdims=True)[:, :, 0]
    s_ii = jnp.sum(q_a, axis=(1, 2), keepdims=True)[:, :, 0] - s_rr
    s_ri = 0.5 * jnp.sum(r_a, axis=(1, 2), keepdims=True)[:, :, 0]

    d_f = jnp.float32(dd)
    inv_dm1 = 1.0 / (d_f - 1.0)
    mr = s_r / d_f
    mi = s_i / d_f
    a = (s_rr - d_f * mr * mr) * inv_dm1 + _EPS
    c = (s_ii - d_f * mi * mi) * inv_dm1 + _EPS
    b = (s_ri - d_f * mr * mi) * inv_dm1

    t = a + c
    s = jnp.sqrt(a * c - b * b)
    denom_inv = jax.lax.rsqrt(t + 2.0 * s) / s
    w00 = (c + s) * denom_inv
    w11 = (a + s) * denom_inv
    w01 = -b * denom_inv

    g0 = gamma_ref[:, 0:1]
    g1 = gamma_ref[:, 1:2]
    g2 = gamma_ref[:, 2:3]
    m00 = (g0 * w00 + g1 * w01)[:, :, None]
    m01 = (g0 * w01 + g1 * w11)[:, :, None]
    m10 = (g1 * w00 + g2 * w01)[:, :, None]
    m11 = (g1 * w01 + g2 * w11)[:, :, None]

    off_e = -(m00 * mr[:, :, None] + m01 * mi[:, :, None])
    off_o = -(m10 * mr[:, :, None] + m11 * mi[:, :, None])

    p = jnp.where(meven8, m00, m11)
    q = jnp.where(meven8, m01, m10)
    o = jnp.where(meven8, off_e, off_o)
    for k in range(rows // 8):
        bk = beta_ref[:, 8 * k:8 * (k + 1), :]
        zk = _sl(k)
        o_ref[:, 8 * k:8 * (k + 1), :] = p * zk + q * _swap(zk) + (o + bk)


def kernel(x, gamma, beta):
    bb, ss, dd, _ = x.shape
    bs = bb * ss
    chunks = dd // 128
    rows = 2 * chunks
    xv = (x.reshape(bs, chunks, 128, 2)
           .transpose(0, 1, 3, 2)
           .reshape(bs, rows, 128))
    betav = (beta.reshape(chunks, 128, 2)
                 .transpose(0, 2, 1)
                 .reshape(1, rows, 128))

    ts = math.gcd(256, math.gcd(bs, dd))
    gperiod = dd // ts

    out = pl.pallas_call(
        _sln_body,
        grid=(bs // ts,),
        in_specs=[
            pl.BlockSpec((ts, rows, 128), lambda j: (j, 0, 0)),
            pl.BlockSpec((ts, 3), lambda j: (j % gperiod, 0)),
            pl.BlockSpec((1, rows, 128), lambda j: (0, 0, 0)),
        ],
        out_specs=pl.BlockSpec((ts, rows, 128), lambda j: (j, 0, 0)),
        out_shape=jax.ShapeDtypeStruct((bs, rows, 128), x.dtype),
        compiler_params=pltpu.CompilerParams(
            dimension_semantics=("parallel",),
        ),
    )(xv, gamma, betav)
    return (out.reshape(bs, chunks, 2, 128)
               .transpose(0, 1, 3, 2)
               .reshape(bb, ss, dd, 2))
